# baseline (speedup 1.0000x reference)
import functools

import jax
import jax.numpy as jnp
from jax import lax
from jax.experimental import pallas as pl
from jax.experimental.pallas import tpu as pltpu
from jax.experimental.pallas import tpu_sc as plsc

VOCAB = 1000000
B = 16384
L = 200
N = B * L

_NC, _NS = 2, 16
_NW = _NC * _NS

_TPAD = 1000448
_RSUB = _TPAD // _NW
_NPIECE = 2
_PIECE = _RSUB // _NPIECE
_PGROUPS = _PIECE // 16
_FPIECE = 3 * _PIECE

_PER_W = N // _NW
_CHUNK = 12800
_NCHUNK = _PER_W // _CHUNK

_mesh = functools.partial(
    plsc.VectorSubcoreMesh, core_axis_name="c", subcore_axis_name="s"
)


def _transform_body(flat_hbm, w_hbm, t_hbm, tab_v, piece_v, w_v):
    wid = lax.axis_index("s") * _NC + lax.axis_index("c")

    pltpu.sync_copy(w_hbm, w_v)
    w = [w_v[pl.ds(16 * k, 16)] for k in range(11)]
    (w00, w01, w02, b10, w10, w11, w12, b11, w20, w21, b2) = w

    i3 = lax.iota(jnp.int32, 16) * 3
    row0 = wid * _RSUB
    for p in range(_NPIECE):
        prow = row0 + p * _PIECE
        pltpu.sync_copy(flat_hbm.at[pl.ds(prow * 3, _FPIECE)], tab_v)

        def tbody(j, carry):
            i0 = i3 + j * 48
            e0 = plsc.load_gather(tab_v, [i0])
            e1 = plsc.load_gather(tab_v, [i0 + 1])
            e2 = plsc.load_gather(tab_v, [i0 + 2])
            h0 = jnp.maximum(e0 * w00 + e1 * w01 + e2 * w02 + b10, 0.0)
            h1 = jnp.maximum(e0 * w10 + e1 * w11 + e2 * w12 + b11, 0.0)
            t = jnp.maximum(h0 * w20 + h1 * w21 + b2, 0.0)
            piece_v[pl.ds(j * 16, 16)] = t
            return carry

        lax.fori_loop(0, _PGROUPS, tbody, 0)
        pltpu.sync_copy(piece_v, t_hbm.at[pl.ds(prow, _PIECE)])


def _gather_body(t_hbm, idx_hbm, out_hbm, idx_v, g_v, sem):
    wid = lax.axis_index("s") * _NC + lax.axis_index("c")
    base = wid * _PER_W

    def gbody(k, carry):
        off = base + k * _CHUNK
        pltpu.sync_copy(idx_hbm.at[pl.ds(off, _CHUNK)], idx_v)
        pltpu.async_copy(t_hbm.at[idx_v], g_v, sem).wait()
        pltpu.sync_copy(g_v, out_hbm.at[pl.ds(off, _CHUNK)])
        return carry

    lax.fori_loop(0, _NCHUNK, gbody, 0)


def kernel(inputs, table, W1, b1, W2, b2):
    flat = jnp.pad(table.reshape(3 * VOCAB), (0, 3 * _TPAD - 3 * VOCAB))
    wvec = jnp.concatenate(
        [W1[0], b1[0:1], W1[1], b1[1:2], W2[0], b2]
    ).astype(jnp.float32)
    wb = jnp.repeat(wvec, 16)

    transform = functools.partial(
        pl.kernel,
        mesh=_mesh(),
        compiler_params=pltpu.CompilerParams(needs_layout_passes=False),
        out_type=jax.ShapeDtypeStruct((_TPAD,), jnp.float32),
        scratch_types=[
            pltpu.VMEM((_FPIECE,), jnp.float32),
            pltpu.VMEM((_PIECE,), jnp.float32),
            pltpu.VMEM((176,), jnp.float32),
        ],
    )(_transform_body)
    t = transform(flat, wb)

    gather = functools.partial(
        pl.kernel,
        mesh=_mesh(),
        out_type=jax.ShapeDtypeStruct((N,), jnp.float32),
        scratch_types=[
            pltpu.VMEM((_CHUNK,), jnp.int32),
            pltpu.VMEM((_CHUNK,), jnp.float32),
            pltpu.SemaphoreType.DMA,
        ],
    )(_gather_body)
    out = gather(t, inputs.reshape(N))
    return out.reshape(B, L, 1)


# speedup vs baseline: 1.0012x; 1.0012x over previous
import functools

import jax
import jax.numpy as jnp
from jax import lax
from jax.experimental import pallas as pl
from jax.experimental.pallas import tpu as pltpu
from jax.experimental.pallas import tpu_sc as plsc

VOCAB = 1000000
B = 16384
L = 200
N = B * L

_NC, _NS = 2, 16
_NW = _NC * _NS

_RSUB = 31264
_RLAST = VOCAB - (_NW - 1) * _RSUB
_NPIECE = 2
_PIECE = _RSUB // _NPIECE
_PLAST = _RLAST // _NPIECE

_PER_W = N // _NW
_CHUNK = 12800
_NCHUNK = _PER_W // _CHUNK

_mesh = functools.partial(
    plsc.VectorSubcoreMesh, core_axis_name="c", subcore_axis_name="s"
)


def _transform_body(flat_hbm, w_hbm, t_hbm, tab_v, piece_v, w_v):
    wid = lax.axis_index("s") * _NC + lax.axis_index("c")

    pltpu.sync_copy(w_hbm, w_v)
    w = [w_v[pl.ds(16 * k, 16)] for k in range(11)]
    (w00, w01, w02, b10, w10, w11, w12, b11, w20, w21, b2) = w

    i3 = lax.iota(jnp.int32, 16) * 3
    row0 = wid * _RSUB

    def piece(prow, n_rows):
        pltpu.sync_copy(flat_hbm.at[pl.ds(prow * 3, 3 * n_rows)], tab_v.at[pl.ds(0, 3 * n_rows)])

        def tbody(j, carry):
            i0 = i3 + j * 48
            e0 = plsc.load_gather(tab_v, [i0])
            e1 = plsc.load_gather(tab_v, [i0 + 1])
            e2 = plsc.load_gather(tab_v, [i0 + 2])
            h0 = jnp.maximum(e0 * w00 + e1 * w01 + e2 * w02 + b10, 0.0)
            h1 = jnp.maximum(e0 * w10 + e1 * w11 + e2 * w12 + b11, 0.0)
            t = jnp.maximum(h0 * w20 + h1 * w21 + b2, 0.0)
            piece_v[pl.ds(j * 16, 16)] = t
            return carry

        lax.fori_loop(0, n_rows // 16, tbody, 0)
        pltpu.sync_copy(piece_v.at[pl.ds(0, n_rows)], t_hbm.at[pl.ds(prow, n_rows)])

    @pl.when(wid < _NW - 1)
    def _main():
        for p in range(_NPIECE):
            piece(row0 + p * _PIECE, _PIECE)

    @pl.when(wid == _NW - 1)
    def _last():
        for p in range(_NPIECE):
            piece(row0 + p * _PLAST, _PLAST)


def _gather_body(t_hbm, idx_hbm, out_hbm, idx_v, g_v, sem):
    wid = lax.axis_index("s") * _NC + lax.axis_index("c")
    base = wid * _PER_W

    def gbody(k, carry):
        off = base + k * _CHUNK
        pltpu.sync_copy(idx_hbm.at[pl.ds(off, _CHUNK)], idx_v)
        pltpu.async_copy(t_hbm.at[idx_v], g_v, sem).wait()
        pltpu.sync_copy(g_v, out_hbm.at[pl.ds(off, _CHUNK)])
        return carry

    lax.fori_loop(0, _NCHUNK, gbody, 0)


def kernel(inputs, table, W1, b1, W2, b2):
    flat = table.reshape(3 * VOCAB)
    wvec = jnp.concatenate(
        [W1[0], b1[0:1], W1[1], b1[1:2], W2[0], b2]
    ).astype(jnp.float32)
    wb = jnp.repeat(wvec, 16)

    transform = functools.partial(
        pl.kernel,
        mesh=_mesh(),
        compiler_params=pltpu.CompilerParams(needs_layout_passes=False),
        out_type=jax.ShapeDtypeStruct((VOCAB,), jnp.float32),
        scratch_types=[
            pltpu.VMEM((3 * _PIECE,), jnp.float32),
            pltpu.VMEM((_PIECE,), jnp.float32),
            pltpu.VMEM((176,), jnp.float32),
        ],
    )(_transform_body)
    t = transform(flat, wb)

    gather = functools.partial(
        pl.kernel,
        mesh=_mesh(),
        out_type=jax.ShapeDtypeStruct((N,), jnp.float32),
        scratch_types=[
            pltpu.VMEM((_CHUNK,), jnp.int32),
            pltpu.VMEM((_CHUNK,), jnp.float32),
            pltpu.SemaphoreType.DMA,
        ],
    )(_gather_body)
    out = gather(t, inputs.reshape(N))
    return out.reshape(B, L, 1)


# speedup vs baseline: 5.1791x; 5.1729x over previous
import functools

import jax
import jax.numpy as jnp
from jax import lax
from jax.experimental import pallas as pl
from jax.experimental.pallas import tpu as pltpu
from jax.experimental.pallas import tpu_sc as plsc

VOCAB = 1000000
B = 16384
L = 200
N = B * L

_NC, _NS = 2, 16
_NW = _NC * _NS

_RSUB = 31264
_RLAST = VOCAB - (_NW - 1) * _RSUB
_NPIECE = 2
_PIECE = _RSUB // _NPIECE
_PLAST = _RLAST // _NPIECE

_PER_W = N // _NW
_CHUNK = 12800
_NCHUNK = _PER_W // _CHUNK

_mesh = functools.partial(
    plsc.VectorSubcoreMesh, core_axis_name="c", subcore_axis_name="s"
)


_TROWS = 8192
_TGRID = 123
_TPAD = _TGRID * _TROWS


def _transform_body(w_ref, tab_ref, out_ref):
    p = tab_ref[...].T
    e0 = p[0:1, :]
    e1 = p[1:2, :]
    e2 = p[2:3, :]
    h0 = jnp.maximum(
        e0 * w_ref[0] + e1 * w_ref[1] + e2 * w_ref[2] + w_ref[3], 0.0)
    h1 = jnp.maximum(
        e0 * w_ref[4] + e1 * w_ref[5] + e2 * w_ref[6] + w_ref[7], 0.0)
    y = jnp.maximum(h0 * w_ref[8] + h1 * w_ref[9] + w_ref[10], 0.0)
    out_ref[...] = y.reshape(_TROWS // 128, 128)


def _gather_body(t_hbm, idx_hbm, out_hbm, idx_v, g_v, sem):
    wid = lax.axis_index("s") * _NC + lax.axis_index("c")
    base = wid * _PER_W

    def gbody(k, carry):
        off = base + k * _CHUNK
        pltpu.sync_copy(idx_hbm.at[pl.ds(off, _CHUNK)], idx_v)
        pltpu.async_copy(t_hbm.at[idx_v], g_v, sem).wait()
        pltpu.sync_copy(g_v, out_hbm.at[pl.ds(off, _CHUNK)])
        return carry

    lax.fori_loop(0, _NCHUNK, gbody, 0)


def kernel(inputs, table, W1, b1, W2, b2):
    wvec = jnp.concatenate(
        [W1[0], b1[0:1], W1[1], b1[1:2], W2[0], b2]
    ).astype(jnp.float32)
    t2 = pl.pallas_call(
        _transform_body,
        grid=(_TGRID,),
        in_specs=[
            pl.BlockSpec(memory_space=pltpu.SMEM),
            pl.BlockSpec((_TROWS, 3), lambda i: (i, 0)),
        ],
        out_specs=pl.BlockSpec((_TROWS // 128, 128), lambda i: (i, 0)),
        out_shape=jax.ShapeDtypeStruct((_TPAD // 128, 128), jnp.float32),
    )(wvec, table)
    t = t2.reshape(_TPAD)

    gather = functools.partial(
        pl.kernel,
        mesh=_mesh(),
        out_type=jax.ShapeDtypeStruct((N,), jnp.float32),
        scratch_types=[
            pltpu.VMEM((_CHUNK,), jnp.int32),
            pltpu.VMEM((_CHUNK,), jnp.float32),
            pltpu.SemaphoreType.DMA,
        ],
    )(_gather_body)
    out = gather(t, inputs.reshape(N))
    return out.reshape(B, L, 1)
